# baseline (speedup 1.0000x reference)
import jax
import jax.numpy as jnp
from jax.experimental import pallas as pl
from jax.experimental.pallas import tpu as pltpu

TOPK = 10
MIN_SIM = 0.8
Q = 32
C = 64
N_ROWS = 1000000
BLK = 16384
NBLK = (N_ROWS + BLK - 1) // BLK
W = 128
NWB = BLK // W
NW = NBLK * NWB
QB = 8
NWIN = QB * TOPK
NEG = -3.0e38


NSPL = 2
SUB = BLK // NSPL
NSUB = (N_ROWS + SUB - 1) // SUB
NWS = SUB // W


def _phase_a(desc_ref, *refs):
    db_refs = refs[:NSPL]
    wm_ref = refs[NSPL]
    i = pl.program_id(0)
    for k in range(NSPL):
        blk = db_refs[k][...]
        sims = jax.lax.dot_general(
            desc_ref[...], blk[:, :C],
            dimension_numbers=(((1,), (1,)), ((), ())),
            preferred_element_type=jnp.float32)
        col = (jax.lax.broadcasted_iota(jnp.int32, (Q, SUB), 1)
               + (i * NSPL + k) * SUB)
        sims = jnp.where(col < N_ROWS, sims, NEG)
        wm = jnp.max(sims.reshape(Q, NWS, W), axis=2)
        wm_ref[:, k * NWS:(k + 1) * NWS] = wm


def _phase_a2(wm_ref, widx_ref):
    cs = wm_ref[...]
    lane = jax.lax.broadcasted_iota(jnp.int32, (Q, NW), 1)
    for k in range(TOPK):
        a = jnp.argmax(cs, axis=1)
        widx_ref[:, k:k + 1] = a[:, None]
        cs = jnp.where(lane == a[:, None], NEG, cs)


def _phase_b(widx_ref, desc_ref, *rest):
    win_refs = rest[:NWIN]
    scores_ref, results_ref = rest[NWIN], rest[NWIN + 1]
    g = pl.program_id(0)
    dble = desc_ref[...].reshape(16, C + 1)
    row = jax.lax.broadcasted_iota(jnp.int32, (QB, W), 0)
    poscol = jax.lax.broadcasted_iota(jnp.int32, (QB, W), 1)
    s_parts = []
    i_parts = []
    for r in range(NWIN):
        wr = win_refs[r][...]
        sfull = jax.lax.dot_general(
            dble, wr,
            dimension_numbers=(((1,), (1,)), ((), ())),
            preferred_element_type=jnp.float32)
        base = widx_ref[g * QB + r // TOPK, r % TOPK] * W
        owner_ok = row == (r // TOPK)
        col_ok = poscol + base < N_ROWS
        s = jnp.where(owner_ok & col_ok, sfull[:QB], NEG)
        ids = jnp.broadcast_to(wr[:, C].reshape(1, W), (QB, W))
        s_parts.append(s)
        i_parts.append(ids)
    cs = jnp.concatenate(s_parts, axis=1)
    ci = jnp.concatenate(i_parts, axis=1)
    lane = jax.lax.broadcasted_iota(jnp.int32, (QB, NWIN * W), 1)
    top_s = []
    top_i = []
    for _ in range(TOPK):
        a = jnp.argmax(cs, axis=1)
        sel = lane == a[:, None]
        top_s.append(jnp.max(cs, axis=1))
        top_i.append(jnp.sum(jnp.where(sel, ci, 0.0), axis=1))
        cs = jnp.where(sel, NEG, cs)
    ts = jnp.stack(top_s, axis=1)
    ti = jnp.stack(top_i, axis=1)

    mask = ts >= MIN_SIM
    maskf = mask.astype(jnp.float32)
    votes = jnp.zeros((QB, TOPK), jnp.float32)
    for j in range(TOPK):
        votes = votes + jnp.where(ti == ti[:, j:j + 1], maskf[:, j:j + 1], 0.0)
    votes = jnp.where(mask, votes, 0.0)
    maxv = jnp.max(votes, axis=1, keepdims=True)
    valid = maxv[:, 0] > 0.0
    cand = jnp.where(mask & (votes == maxv), ti, 3.0e38)
    maj = jnp.min(cand, axis=1)
    res_f = jnp.where(valid, maj, -1.0)
    match = mask & (ti == res_f[:, None])
    sim_sc = jnp.max(jnp.where(match, ts, 0.0), axis=1)
    scores_ref[...] = sim_sc.reshape(QB, 1, 1)
    results_ref[...] = res_f.reshape(QB, 1, 1).astype(jnp.int32)


@jax.jit
def _run(descriptors, places_db):
    wm = pl.pallas_call(
        _phase_a,
        grid=(NBLK,),
        in_specs=[pl.BlockSpec((Q, C), lambda i: (0, 0))]
        + [
            pl.BlockSpec(
                (SUB, C + 1),
                (lambda i, _k=k:
                 (jnp.minimum(i * NSPL + _k, NSUB - 1), 0)))
            for k in range(NSPL)
        ],
        out_specs=pl.BlockSpec((Q, NWB), lambda i: (0, i)),
        out_shape=jax.ShapeDtypeStruct((Q, NW), jnp.float32),
    )(descriptors, *([places_db] * NSPL))

    widx = pl.pallas_call(
        _phase_a2,
        in_specs=[pl.BlockSpec((Q, NW), lambda: (0, 0))],
        out_specs=pl.BlockSpec((Q, TOPK), lambda: (0, 0)),
        out_shape=jax.ShapeDtypeStruct((Q, TOPK), jnp.int32),
    )(wm)

    desc_ext = jnp.concatenate(
        [descriptors, jnp.zeros((Q, 1), jnp.float32)], axis=1)
    e64 = jnp.zeros((1, C + 1), jnp.float32).at[0, C].set(1.0)
    ngrp = Q // QB
    desc_grp = jnp.concatenate(
        [desc_ext.reshape(ngrp, QB, C + 1),
         jnp.broadcast_to(e64[None], (ngrp, 1, C + 1)),
         jnp.zeros((ngrp, 16 - QB - 1, C + 1), jnp.float32)],
        axis=1)
    db_specs = [
        pl.BlockSpec(
            (W, C + 1),
            (lambda g, widx_ref, _r=r:
             (widx_ref[g * QB + _r // TOPK, _r % TOPK], 0)))
        for r in range(NWIN)
    ]
    scores, results = pl.pallas_call(
        _phase_b,
        grid_spec=pltpu.PrefetchScalarGridSpec(
            num_scalar_prefetch=1,
            grid=(Q // QB,),
            in_specs=[
                pl.BlockSpec((1, 16, C + 1), lambda g, widx_ref: (g, 0, 0)),
            ] + db_specs,
            out_specs=[
                pl.BlockSpec((QB, 1, 1), lambda g, widx_ref: (g, 0, 0)),
                pl.BlockSpec((QB, 1, 1), lambda g, widx_ref: (g, 0, 0)),
            ],
        ),
        out_shape=[
            jax.ShapeDtypeStruct((Q, 1, 1), jnp.float32),
            jax.ShapeDtypeStruct((Q, 1, 1), jnp.int32),
        ],
    )(widx, desc_grp, *([places_db] * NWIN))
    return scores.reshape(Q), results.reshape(Q)


def kernel(final_boxes, descriptors, places_db):
    sim_scores, results = _run(descriptors, places_db)
    return (final_boxes, sim_scores, results)


# speedup vs baseline: 1.0094x; 1.0094x over previous
import jax
import jax.numpy as jnp
from jax.experimental import pallas as pl
from jax.experimental.pallas import tpu as pltpu

TOPK = 10
MIN_SIM = 0.8
Q = 32
C = 64
N_ROWS = 1000000
BLK = 32768
NBLK = (N_ROWS + BLK - 1) // BLK
W = 128
NWB = BLK // W
NW = NBLK * NWB
QB = 8
NWIN = QB * TOPK
NEG = -3.0e38


NSPL = 2
SUB = BLK // NSPL
NSUB = (N_ROWS + SUB - 1) // SUB
NWS = SUB // W


def _phase_a(desc_ref, *refs):
    db_refs = refs[:NSPL]
    wm_ref = refs[NSPL]
    i = pl.program_id(0)
    for k in range(NSPL):
        blk = db_refs[k][...]
        sims = jax.lax.dot_general(
            desc_ref[...], blk[:, :C],
            dimension_numbers=(((1,), (1,)), ((), ())),
            preferred_element_type=jnp.float32)
        col = (jax.lax.broadcasted_iota(jnp.int32, (Q, SUB), 1)
               + (i * NSPL + k) * SUB)
        sims = jnp.where(col < N_ROWS, sims, NEG)
        wm = jnp.max(sims.reshape(Q, NWS, W), axis=2)
        wm_ref[:, k * NWS:(k + 1) * NWS] = wm


def _phase_a2(wm_ref, widx_ref):
    cs = wm_ref[...]
    lane = jax.lax.broadcasted_iota(jnp.int32, (Q, NW), 1)
    for k in range(TOPK):
        a = jnp.argmax(cs, axis=1)
        widx_ref[:, k:k + 1] = a[:, None]
        cs = jnp.where(lane == a[:, None], NEG, cs)


def _phase_b(widx_ref, desc_ref, *rest):
    win_refs = rest[:NWIN]
    scores_ref, results_ref = rest[NWIN], rest[NWIN + 1]
    g = pl.program_id(0)
    dble = desc_ref[...].reshape(16, C + 1)
    row = jax.lax.broadcasted_iota(jnp.int32, (QB, W), 0)
    poscol = jax.lax.broadcasted_iota(jnp.int32, (QB, W), 1)
    s_parts = []
    i_parts = []
    for r in range(NWIN):
        wr = win_refs[r][...]
        sfull = jax.lax.dot_general(
            dble, wr,
            dimension_numbers=(((1,), (1,)), ((), ())),
            preferred_element_type=jnp.float32)
        base = widx_ref[g * QB + r // TOPK, r % TOPK] * W
        owner_ok = row == (r // TOPK)
        col_ok = poscol + base < N_ROWS
        s = jnp.where(owner_ok & col_ok, sfull[:QB], NEG)
        ids = jnp.broadcast_to(wr[:, C].reshape(1, W), (QB, W))
        s_parts.append(s)
        i_parts.append(ids)
    cs = jnp.concatenate(s_parts, axis=1)
    ci = jnp.concatenate(i_parts, axis=1)
    lane = jax.lax.broadcasted_iota(jnp.int32, (QB, NWIN * W), 1)
    top_s = []
    top_i = []
    for _ in range(TOPK):
        a = jnp.argmax(cs, axis=1)
        sel = lane == a[:, None]
        top_s.append(jnp.max(cs, axis=1))
        top_i.append(jnp.sum(jnp.where(sel, ci, 0.0), axis=1))
        cs = jnp.where(sel, NEG, cs)
    ts = jnp.stack(top_s, axis=1)
    ti = jnp.stack(top_i, axis=1)

    mask = ts >= MIN_SIM
    maskf = mask.astype(jnp.float32)
    votes = jnp.zeros((QB, TOPK), jnp.float32)
    for j in range(TOPK):
        votes = votes + jnp.where(ti == ti[:, j:j + 1], maskf[:, j:j + 1], 0.0)
    votes = jnp.where(mask, votes, 0.0)
    maxv = jnp.max(votes, axis=1, keepdims=True)
    valid = maxv[:, 0] > 0.0
    cand = jnp.where(mask & (votes == maxv), ti, 3.0e38)
    maj = jnp.min(cand, axis=1)
    res_f = jnp.where(valid, maj, -1.0)
    match = mask & (ti == res_f[:, None])
    sim_sc = jnp.max(jnp.where(match, ts, 0.0), axis=1)
    scores_ref[...] = sim_sc.reshape(QB, 1, 1)
    results_ref[...] = res_f.reshape(QB, 1, 1).astype(jnp.int32)


@jax.jit
def _run(descriptors, places_db):
    wm = pl.pallas_call(
        _phase_a,
        grid=(NBLK,),
        in_specs=[pl.BlockSpec((Q, C), lambda i: (0, 0))]
        + [
            pl.BlockSpec(
                (SUB, C + 1),
                (lambda i, _k=k:
                 (jnp.minimum(i * NSPL + _k, NSUB - 1), 0)))
            for k in range(NSPL)
        ],
        out_specs=pl.BlockSpec((Q, NWB), lambda i: (0, i)),
        out_shape=jax.ShapeDtypeStruct((Q, NW), jnp.float32),
    )(descriptors, *([places_db] * NSPL))

    widx = pl.pallas_call(
        _phase_a2,
        in_specs=[pl.BlockSpec((Q, NW), lambda: (0, 0))],
        out_specs=pl.BlockSpec((Q, TOPK), lambda: (0, 0)),
        out_shape=jax.ShapeDtypeStruct((Q, TOPK), jnp.int32),
    )(wm)

    desc_ext = jnp.concatenate(
        [descriptors, jnp.zeros((Q, 1), jnp.float32)], axis=1)
    e64 = jnp.zeros((1, C + 1), jnp.float32).at[0, C].set(1.0)
    ngrp = Q // QB
    desc_grp = jnp.concatenate(
        [desc_ext.reshape(ngrp, QB, C + 1),
         jnp.broadcast_to(e64[None], (ngrp, 1, C + 1)),
         jnp.zeros((ngrp, 16 - QB - 1, C + 1), jnp.float32)],
        axis=1)
    db_specs = [
        pl.BlockSpec(
            (W, C + 1),
            (lambda g, widx_ref, _r=r:
             (widx_ref[g * QB + _r // TOPK, _r % TOPK], 0)))
        for r in range(NWIN)
    ]
    scores, results = pl.pallas_call(
        _phase_b,
        grid_spec=pltpu.PrefetchScalarGridSpec(
            num_scalar_prefetch=1,
            grid=(Q // QB,),
            in_specs=[
                pl.BlockSpec((1, 16, C + 1), lambda g, widx_ref: (g, 0, 0)),
            ] + db_specs,
            out_specs=[
                pl.BlockSpec((QB, 1, 1), lambda g, widx_ref: (g, 0, 0)),
                pl.BlockSpec((QB, 1, 1), lambda g, widx_ref: (g, 0, 0)),
            ],
        ),
        out_shape=[
            jax.ShapeDtypeStruct((Q, 1, 1), jnp.float32),
            jax.ShapeDtypeStruct((Q, 1, 1), jnp.int32),
        ],
    )(widx, desc_grp, *([places_db] * NWIN))
    return scores.reshape(Q), results.reshape(Q)


def kernel(final_boxes, descriptors, places_db):
    sim_scores, results = _run(descriptors, places_db)
    return (final_boxes, sim_scores, results)


# speedup vs baseline: 1.0123x; 1.0029x over previous
import jax
import jax.numpy as jnp
from jax.experimental import pallas as pl
from jax.experimental.pallas import tpu as pltpu

TOPK = 10
MIN_SIM = 0.8
Q = 32
C = 64
N_ROWS = 1000000
BLK = 32768
NBLK = (N_ROWS + BLK - 1) // BLK
W = 128
NWB = BLK // W
NW = NBLK * NWB
QB = 8
NWIN = QB * TOPK
NEG = -3.0e38


NSPL = 4
SUB = BLK // NSPL
NSUB = (N_ROWS + SUB - 1) // SUB
NWS = SUB // W


def _phase_a(desc_ref, *refs):
    db_refs = refs[:NSPL]
    wm_ref = refs[NSPL]
    i = pl.program_id(0)
    for k in range(NSPL):
        blk = db_refs[k][...]
        sims = jax.lax.dot_general(
            desc_ref[...], blk[:, :C],
            dimension_numbers=(((1,), (1,)), ((), ())),
            preferred_element_type=jnp.float32)
        col = (jax.lax.broadcasted_iota(jnp.int32, (Q, SUB), 1)
               + (i * NSPL + k) * SUB)
        sims = jnp.where(col < N_ROWS, sims, NEG)
        wm = jnp.max(sims.reshape(Q, NWS, W), axis=2)
        wm_ref[:, k * NWS:(k + 1) * NWS] = wm


def _phase_a2(wm_ref, widx_ref):
    cs = wm_ref[...]
    lane = jax.lax.broadcasted_iota(jnp.int32, (Q, NW), 1)
    for k in range(TOPK):
        a = jnp.argmax(cs, axis=1)
        widx_ref[:, k:k + 1] = a[:, None]
        cs = jnp.where(lane == a[:, None], NEG, cs)


def _phase_b(widx_ref, desc_ref, *rest):
    win_refs = rest[:NWIN]
    scores_ref, results_ref = rest[NWIN], rest[NWIN + 1]
    g = pl.program_id(0)
    dble = desc_ref[...].reshape(16, C + 1)
    row = jax.lax.broadcasted_iota(jnp.int32, (QB, W), 0)
    poscol = jax.lax.broadcasted_iota(jnp.int32, (QB, W), 1)
    s_parts = []
    i_parts = []
    for r in range(NWIN):
        wr = win_refs[r][...]
        sfull = jax.lax.dot_general(
            dble, wr,
            dimension_numbers=(((1,), (1,)), ((), ())),
            preferred_element_type=jnp.float32)
        base = widx_ref[g * QB + r // TOPK, r % TOPK] * W
        owner_ok = row == (r // TOPK)
        col_ok = poscol + base < N_ROWS
        s = jnp.where(owner_ok & col_ok, sfull[:QB], NEG)
        ids = jnp.broadcast_to(wr[:, C].reshape(1, W), (QB, W))
        s_parts.append(s)
        i_parts.append(ids)
    cs = jnp.concatenate(s_parts, axis=1)
    ci = jnp.concatenate(i_parts, axis=1)
    lane = jax.lax.broadcasted_iota(jnp.int32, (QB, NWIN * W), 1)
    top_s = []
    top_i = []
    for _ in range(TOPK):
        a = jnp.argmax(cs, axis=1)
        sel = lane == a[:, None]
        top_s.append(jnp.max(cs, axis=1))
        top_i.append(jnp.sum(jnp.where(sel, ci, 0.0), axis=1))
        cs = jnp.where(sel, NEG, cs)
    ts = jnp.stack(top_s, axis=1)
    ti = jnp.stack(top_i, axis=1)

    mask = ts >= MIN_SIM
    maskf = mask.astype(jnp.float32)
    votes = jnp.zeros((QB, TOPK), jnp.float32)
    for j in range(TOPK):
        votes = votes + jnp.where(ti == ti[:, j:j + 1], maskf[:, j:j + 1], 0.0)
    votes = jnp.where(mask, votes, 0.0)
    maxv = jnp.max(votes, axis=1, keepdims=True)
    valid = maxv[:, 0] > 0.0
    cand = jnp.where(mask & (votes == maxv), ti, 3.0e38)
    maj = jnp.min(cand, axis=1)
    res_f = jnp.where(valid, maj, -1.0)
    match = mask & (ti == res_f[:, None])
    sim_sc = jnp.max(jnp.where(match, ts, 0.0), axis=1)
    scores_ref[...] = sim_sc.reshape(QB, 1, 1)
    results_ref[...] = res_f.reshape(QB, 1, 1).astype(jnp.int32)


@jax.jit
def _run(descriptors, places_db):
    wm = pl.pallas_call(
        _phase_a,
        grid=(NBLK,),
        in_specs=[pl.BlockSpec((Q, C), lambda i: (0, 0))]
        + [
            pl.BlockSpec(
                (SUB, C + 1),
                (lambda i, _k=k:
                 (jnp.minimum(i * NSPL + _k, NSUB - 1), 0)))
            for k in range(NSPL)
        ],
        out_specs=pl.BlockSpec((Q, NWB), lambda i: (0, i)),
        out_shape=jax.ShapeDtypeStruct((Q, NW), jnp.float32),
    )(descriptors, *([places_db] * NSPL))

    widx = pl.pallas_call(
        _phase_a2,
        in_specs=[pl.BlockSpec((Q, NW), lambda: (0, 0))],
        out_specs=pl.BlockSpec((Q, TOPK), lambda: (0, 0)),
        out_shape=jax.ShapeDtypeStruct((Q, TOPK), jnp.int32),
    )(wm)

    desc_ext = jnp.concatenate(
        [descriptors, jnp.zeros((Q, 1), jnp.float32)], axis=1)
    e64 = jnp.zeros((1, C + 1), jnp.float32).at[0, C].set(1.0)
    ngrp = Q // QB
    desc_grp = jnp.concatenate(
        [desc_ext.reshape(ngrp, QB, C + 1),
         jnp.broadcast_to(e64[None], (ngrp, 1, C + 1)),
         jnp.zeros((ngrp, 16 - QB - 1, C + 1), jnp.float32)],
        axis=1)
    db_specs = [
        pl.BlockSpec(
            (W, C + 1),
            (lambda g, widx_ref, _r=r:
             (widx_ref[g * QB + _r // TOPK, _r % TOPK], 0)))
        for r in range(NWIN)
    ]
    scores, results = pl.pallas_call(
        _phase_b,
        grid_spec=pltpu.PrefetchScalarGridSpec(
            num_scalar_prefetch=1,
            grid=(Q // QB,),
            in_specs=[
                pl.BlockSpec((1, 16, C + 1), lambda g, widx_ref: (g, 0, 0)),
            ] + db_specs,
            out_specs=[
                pl.BlockSpec((QB, 1, 1), lambda g, widx_ref: (g, 0, 0)),
                pl.BlockSpec((QB, 1, 1), lambda g, widx_ref: (g, 0, 0)),
            ],
        ),
        out_shape=[
            jax.ShapeDtypeStruct((Q, 1, 1), jnp.float32),
            jax.ShapeDtypeStruct((Q, 1, 1), jnp.int32),
        ],
    )(widx, desc_grp, *([places_db] * NWIN))
    return scores.reshape(Q), results.reshape(Q)


def kernel(final_boxes, descriptors, places_db):
    sim_scores, results = _run(descriptors, places_db)
    return (final_boxes, sim_scores, results)
